# baseline (speedup 1.0000x reference)
import functools
import math

import jax
import jax.numpy as jnp
from jax.experimental import pallas as pl
from jax.experimental.pallas import tpu as pltpu

_ANCHOR_RATIO = 0.1
_MIN_ANCHORS = 1


def _body(patches_ref, out_ref, acc, *, n_chunks, n):
    ic = pl.program_id(1)

    @pl.when(ic == 0)
    def _zero():
        acc[...] = jnp.zeros_like(acc)

    x = patches_ref[0]
    acc[...] += jnp.sum(x, axis=0)

    @pl.when(ic == n_chunks - 1)
    def _finish():
        out_ref[0] = acc[...] * (1.0 / n)


def kernel(patches, adp):
    b, n, p, d = patches.shape
    if p == 0:
        return jnp.zeros((b * n, 0, d), dtype=patches.dtype)
    k = min(max(_MIN_ANCHORS, int(math.ceil(p * _ANCHOR_RATIO))), p)

    importance = adp.mean(axis=0)
    norms = jnp.linalg.norm(patches, axis=-1)
    scores = jnp.einsum('bnp,n->bp', norms, importance)
    _, topk_idx = jax.lax.top_k(scores, k)

    rows = p * d // 128
    flat = patches.reshape(b, n, rows, 128)
    nc = 16
    n_chunks = n // nc
    mean_flat = pl.pallas_call(
        functools.partial(_body, n_chunks=n_chunks, n=n),
        grid=(b, n_chunks),
        in_specs=[
            pl.BlockSpec((1, nc, rows, 128), lambda ib, ic: (ib, ic, 0, 0)),
        ],
        out_specs=pl.BlockSpec((1, rows, 128), lambda ib, ic: (ib, 0, 0)),
        out_shape=jax.ShapeDtypeStruct((b, rows, 128), patches.dtype),
        scratch_shapes=[pltpu.VMEM((rows, 128), jnp.float32)],
        compiler_params=pltpu.CompilerParams(
            dimension_semantics=("parallel", "arbitrary"),
        ),
    )(flat)
    mean_patches = mean_flat.reshape(b, p, d)
    anchors = jnp.take_along_axis(mean_patches, topk_idx[:, :, None], axis=1)
    anchors = jnp.broadcast_to(anchors[:, None, :, :], (b, n, k, d))
    return anchors.reshape(b * n, k, d)
